# baseline (speedup 1.0000x reference)
import jax
import jax.numpy as jnp
from jax.experimental import pallas as pl
from jax.experimental.pallas import tpu as pltpu

_B, _L, _D = 4096, 128, 64


_NWAY = 8


def _copy_body(p_ref, *refs):
    del p_ref
    xs, o_ref = refs[:_NWAY], refs[_NWAY]
    for k, x_ref in enumerate(xs):
        o_ref[:, k] = x_ref[:, 0]


def _permute_rows(x, p_array):
    nb = x.shape[0]
    x4 = x.reshape(nb, _L, 1, _D)

    nh = nb // 2

    def in_map(k):
        return lambda i, j, p: (j, p[_NWAY * i + k], 0, 0)

    out = pl.pallas_call(
        _copy_body,
        grid_spec=pltpu.PrefetchScalarGridSpec(
            num_scalar_prefetch=1,
            grid=(_L // _NWAY, 2),
            in_specs=[pl.BlockSpec((nh, 1, 1, _D), in_map(k))
                      for k in range(_NWAY)],
            out_specs=pl.BlockSpec((nh, _NWAY, 1, _D),
                                   lambda i, j, p: (j, i, 0, 0)),
        ),
        out_shape=jax.ShapeDtypeStruct(x4.shape, jnp.float32),
    )(p_array, *([x4] * _NWAY))
    return out.reshape(nb, _L, _D)


def kernel(inputs, p_array):
    return _permute_rows(inputs, p_array)


# speedup vs baseline: 1.0098x; 1.0098x over previous
import jax
import jax.numpy as jnp
from jax.experimental import pallas as pl
from jax.experimental.pallas import tpu as pltpu

_B, _L, _D = 4096, 128, 64


_NWAY = 16


def _copy_body(p_ref, *refs):
    del p_ref
    xs, o_ref = refs[:_NWAY], refs[_NWAY]
    for k, x_ref in enumerate(xs):
        o_ref[:, k] = x_ref[:, 0]


def _permute_rows(x, p_array):
    nb = x.shape[0]
    x4 = x.reshape(nb, _L, 1, _D)

    nh = nb // 4

    def in_map(k):
        return lambda i, j, p: (j, p[_NWAY * i + k], 0, 0)

    out = pl.pallas_call(
        _copy_body,
        grid_spec=pltpu.PrefetchScalarGridSpec(
            num_scalar_prefetch=1,
            grid=(_L // _NWAY, 4),
            in_specs=[pl.BlockSpec((nh, 1, 1, _D), in_map(k))
                      for k in range(_NWAY)],
            out_specs=pl.BlockSpec((nh, _NWAY, 1, _D),
                                   lambda i, j, p: (j, i, 0, 0)),
        ),
        out_shape=jax.ShapeDtypeStruct(x4.shape, jnp.float32),
    )(p_array, *([x4] * _NWAY))
    return out.reshape(nb, _L, _D)


def kernel(inputs, p_array):
    return _permute_rows(inputs, p_array)


# speedup vs baseline: 1.0133x; 1.0034x over previous
import jax
import jax.numpy as jnp
from jax.experimental import pallas as pl
from jax.experimental.pallas import tpu as pltpu

_B, _L, _D = 4096, 128, 64
_NWAY = 4


def _copy_body(p_ref, *refs):
    del p_ref
    xs, o_ref = refs[:_NWAY], refs[_NWAY]
    for k, x_ref in enumerate(xs):
        o_ref[:, k] = x_ref[:, 0]


def _permute_rows(x, p_array):
    nb = x.shape[0]
    x4 = x.reshape(nb, _L, 1, _D)

    def in_map(k):
        return lambda i, p: (0, p[_NWAY * i + k], 0, 0)

    out = pl.pallas_call(
        _copy_body,
        grid_spec=pltpu.PrefetchScalarGridSpec(
            num_scalar_prefetch=1,
            grid=(_L // _NWAY,),
            in_specs=[pl.BlockSpec((nb, 1, 1, _D), in_map(k))
                      for k in range(_NWAY)],
            out_specs=pl.BlockSpec((nb, _NWAY, 1, _D),
                                   lambda i, p: (0, i, 0, 0)),
        ),
        out_shape=jax.ShapeDtypeStruct(x4.shape, jnp.float32),
    )(p_array, *([x4] * _NWAY))
    return out.reshape(nb, _L, _D)


def kernel(inputs, p_array):
    return _permute_rows(inputs, p_array)
